# Initial kernel scaffold; baseline (speedup 1.0000x reference)
import jax
import jax.numpy as jnp
from jax.experimental import pallas as pl


def kernel(x, table, pos_encoding):
    raise NotImplementedError("write your pallas kernel here")


# speedup vs baseline: 2.3593x; 2.3593x over previous
import functools

import jax
import jax.numpy as jnp
from jax import lax
from jax.experimental import pallas as pl
from jax.experimental.pallas import tpu as pltpu
from jax.experimental.pallas import tpu_sc as plsc

D = 128
SCALE = float(D) ** 0.5
NW = 32
CHUNK = 100
LANES = 16


def _body(x_hbm, table_hbm, pos_hbm, out_hbm,
          idx_v, pos_v, in0, in1, out0, out1, gs0, gs1, ss0, ss1):
    n_chunks_w = idx_v.shape[0]
    seq = pos_v.shape[0]
    wid = lax.axis_index("s") * 2 + lax.axis_index("c")
    gbase = wid * (n_chunks_w * CHUNK)

    pltpu.sync_copy(x_hbm.at[pl.ds(wid * n_chunks_w, n_chunks_w)], idx_v)
    pltpu.sync_copy(pos_hbm, pos_v)

    pltpu.async_copy(table_hbm.at[idx_v.at[0]], in0, gs0)
    pltpu.async_copy(table_hbm.at[idx_v.at[1]], in1, gs1)

    n_pairs = n_chunks_w // 2

    def pair_body(t, carry):
        for p, (inb, outb, gs, ss) in enumerate(
                ((in0, out0, gs0, ss0), (in1, out1, gs1, ss1))):
            j = 2 * t + p
            pltpu.make_async_copy(table_hbm.at[idx_v.at[j]], inb, gs).wait()
            @pl.when(t > 0)
            def _wait_prev_scatter():
                pltpu.make_async_copy(
                    outb, out_hbm.at[pl.ds(gbase + (j - 2) * CHUNK, CHUNK)],
                    ss).wait()

            poff = p * CHUNK

            def row_body(r, c2):
                for c in range(D // LANES):
                    sl = pl.ds(c * LANES, LANES)
                    outb[r, sl] = inb[r, sl] * SCALE + pos_v[poff + r, sl]
                return c2
            lax.fori_loop(0, CHUNK, row_body, 0, unroll=2)

            @pl.when(t < n_pairs - 1)
            def _next_gather():
                pltpu.async_copy(table_hbm.at[idx_v.at[j + 2]], inb, gs)

            pltpu.async_copy(
                outb, out_hbm.at[pl.ds(gbase + j * CHUNK, CHUNK)], ss)
        return carry

    lax.fori_loop(0, n_pairs, pair_body, 0)

    last = n_chunks_w - 2
    pltpu.make_async_copy(
        out0, out_hbm.at[pl.ds(gbase + last * CHUNK, CHUNK)], ss0).wait()
    pltpu.make_async_copy(
        out1, out_hbm.at[pl.ds(gbase + (last + 1) * CHUNK, CHUNK)], ss1).wait()


def kernel(x, table, pos_encoding):
    B, S = x.shape
    N = B * S
    n_chunks = N // CHUNK
    x2 = x.reshape(n_chunks, CHUNK)
    seq = pos_encoding.shape[0]

    mesh = plsc.VectorSubcoreMesh(core_axis_name="c", subcore_axis_name="s")
    run = pl.kernel(
        _body,
        out_type=jax.ShapeDtypeStruct((N, D), jnp.float32),
        mesh=mesh,
        compiler_params=pltpu.CompilerParams(use_tc_tiling_on_sc=False),
        scratch_types=[
            pltpu.VMEM((n_chunks // NW, CHUNK), jnp.int32),
            pltpu.VMEM((seq, D), jnp.float32),
            pltpu.VMEM((CHUNK, D), jnp.float32),
            pltpu.VMEM((CHUNK, D), jnp.float32),
            pltpu.VMEM((CHUNK, D), jnp.float32),
            pltpu.VMEM((CHUNK, D), jnp.float32),
            pltpu.SemaphoreType.DMA,
            pltpu.SemaphoreType.DMA,
            pltpu.SemaphoreType.DMA,
            pltpu.SemaphoreType.DMA,
        ],
    )
    out = run(x2, table, pos_encoding)
    return out.reshape(B, S, D)


# speedup vs baseline: 6.7757x; 2.8719x over previous
import functools

import jax
import jax.numpy as jnp
from jax import lax
from jax.experimental import pallas as pl
from jax.experimental.pallas import tpu as pltpu
from jax.experimental.pallas import tpu_sc as plsc

D = 128
SCALE = float(D) ** 0.5
NW = 32
CHUNK = 100
LANES = 16


def _body(x_hbm, table_hbm, pos_hbm, out_hbm,
          idx_v, pos_v, in0, in1, out0, out1, gs0, gs1, ss0, ss1):
    n_chunks_w = idx_v.shape[0]
    seq = pos_v.shape[0]
    wid = lax.axis_index("s") * 2 + lax.axis_index("c")
    gbase = wid * (n_chunks_w * CHUNK)

    pltpu.sync_copy(x_hbm.at[pl.ds(wid * n_chunks_w, n_chunks_w)], idx_v)
    pltpu.sync_copy(pos_hbm, pos_v)

    pltpu.async_copy(table_hbm.at[idx_v.at[0]], in0, gs0)
    pltpu.async_copy(table_hbm.at[idx_v.at[1]], in1, gs1)

    n_pairs = n_chunks_w // 2

    def pair_body(t, carry):
        for p, (inb, outb, gs, ss) in enumerate(
                ((in0, out0, gs0, ss0), (in1, out1, gs1, ss1))):
            j = 2 * t + p
            pltpu.make_async_copy(table_hbm.at[idx_v.at[j]], inb, gs).wait()
            @pl.when(t > 0)
            def _wait_prev_scatter():
                pltpu.make_async_copy(
                    outb, out_hbm.at[pl.ds(gbase + (j - 2) * CHUNK, CHUNK)],
                    ss).wait()

            poff = p * CHUNK

            @plsc.parallel_loop(0, CHUNK, step=1, unroll=4)
            def _compute(r):
                for c in range(D // LANES):
                    sl = pl.ds(c * LANES, LANES)
                    outb[r, sl] = inb[r, sl] * SCALE + pos_v[poff + r, sl]

            @pl.when(t < n_pairs - 1)
            def _next_gather():
                pltpu.async_copy(table_hbm.at[idx_v.at[j + 2]], inb, gs)

            pltpu.async_copy(
                outb, out_hbm.at[pl.ds(gbase + j * CHUNK, CHUNK)], ss)
        return carry

    lax.fori_loop(0, n_pairs, pair_body, 0)

    last = n_chunks_w - 2
    pltpu.make_async_copy(
        out0, out_hbm.at[pl.ds(gbase + last * CHUNK, CHUNK)], ss0).wait()
    pltpu.make_async_copy(
        out1, out_hbm.at[pl.ds(gbase + (last + 1) * CHUNK, CHUNK)], ss1).wait()


def kernel(x, table, pos_encoding):
    B, S = x.shape
    N = B * S
    n_chunks = N // CHUNK
    x2 = x.reshape(n_chunks, CHUNK)
    seq = pos_encoding.shape[0]

    mesh = plsc.VectorSubcoreMesh(core_axis_name="c", subcore_axis_name="s")
    run = pl.kernel(
        _body,
        out_type=jax.ShapeDtypeStruct((N, D), jnp.float32),
        mesh=mesh,
        compiler_params=pltpu.CompilerParams(use_tc_tiling_on_sc=False),
        scratch_types=[
            pltpu.VMEM((n_chunks // NW, CHUNK), jnp.int32),
            pltpu.VMEM((seq, D), jnp.float32),
            pltpu.VMEM((CHUNK, D), jnp.float32),
            pltpu.VMEM((CHUNK, D), jnp.float32),
            pltpu.VMEM((CHUNK, D), jnp.float32),
            pltpu.VMEM((CHUNK, D), jnp.float32),
            pltpu.SemaphoreType.DMA,
            pltpu.SemaphoreType.DMA,
            pltpu.SemaphoreType.DMA,
            pltpu.SemaphoreType.DMA,
        ],
    )
    out = run(x2, table, pos_encoding)
    return out.reshape(B, S, D)


# speedup vs baseline: 7.3998x; 1.0921x over previous
import jax
import jax.numpy as jnp
from jax import lax
from jax.experimental import pallas as pl
from jax.experimental.pallas import tpu as pltpu
from jax.experimental.pallas import tpu_sc as plsc

D = 128
SCALE = float(D) ** 0.5
NW = 32
CHUNK = 100
LANES = 16
NBUF = 4


def _body(x_hbm, table_hbm, pos_hbm, out_hbm,
          idx_v, pos_v, ins, outs, gsems, ssems):
    n_chunks_w = idx_v.shape[0]
    wid = lax.axis_index("s") * 2 + lax.axis_index("c")
    gbase = wid * (n_chunks_w * CHUNK)

    pltpu.sync_copy(x_hbm.at[pl.ds(wid * n_chunks_w, n_chunks_w)], idx_v)
    pltpu.sync_copy(pos_hbm, pos_v)

    for b in range(NBUF):
        pltpu.async_copy(table_hbm.at[idx_v.at[b]], ins[b], gsems[b])

    n_iters = n_chunks_w // NBUF

    def iter_body(t, carry):
        for p in range(NBUF):
            j = NBUF * t + p
            inb, gs = ins[p], gsems[p]
            outb, ss = outs[p % 2], ssems[p % 2]
            pltpu.make_async_copy(table_hbm.at[idx_v.at[j]], inb, gs).wait()
            @pl.when(j > 1)
            def _wait_prev_scatter():
                pltpu.make_async_copy(
                    outb, out_hbm.at[pl.ds(gbase + (j - 2) * CHUNK, CHUNK)],
                    ss).wait()

            poff = (p % 2) * CHUNK

            @plsc.parallel_loop(0, CHUNK, step=1, unroll=4)
            def _compute(r):
                for c in range(D // LANES):
                    sl = pl.ds(c * LANES, LANES)
                    outb[r, sl] = inb[r, sl] * SCALE + pos_v[poff + r, sl]

            @pl.when(t < n_iters - 1)
            def _next_gather():
                pltpu.async_copy(table_hbm.at[idx_v.at[j + NBUF]], inb, gs)

            pltpu.async_copy(
                outb, out_hbm.at[pl.ds(gbase + j * CHUNK, CHUNK)], ss)
        return carry

    lax.fori_loop(0, n_iters, iter_body, 0)

    last = n_chunks_w - 2
    for q in range(2):
        pltpu.make_async_copy(
            outs[q], out_hbm.at[pl.ds(gbase + (last + q) * CHUNK, CHUNK)],
            ssems[q]).wait()


def kernel(x, table, pos_encoding):
    B, S = x.shape
    N = B * S
    n_chunks = N // CHUNK
    x2 = x.reshape(n_chunks, CHUNK)
    seq = pos_encoding.shape[0]

    mesh = plsc.VectorSubcoreMesh(core_axis_name="c", subcore_axis_name="s")

    def body(x_hbm, table_hbm, pos_hbm, out_hbm,
             idx_v, pos_v, in0, in1, in2, in3, out0, out1,
             gs0, gs1, gs2, gs3, ss0, ss1):
        _body(x_hbm, table_hbm, pos_hbm, out_hbm, idx_v, pos_v,
              (in0, in1, in2, in3), (out0, out1),
              (gs0, gs1, gs2, gs3), (ss0, ss1))

    run = pl.kernel(
        body,
        out_type=jax.ShapeDtypeStruct((N, D), jnp.float32),
        mesh=mesh,
        compiler_params=pltpu.CompilerParams(use_tc_tiling_on_sc=False),
        scratch_types=[
            pltpu.VMEM((n_chunks // NW, CHUNK), jnp.int32),
            pltpu.VMEM((seq, D), jnp.float32),
            pltpu.VMEM((CHUNK, D), jnp.float32),
            pltpu.VMEM((CHUNK, D), jnp.float32),
            pltpu.VMEM((CHUNK, D), jnp.float32),
            pltpu.VMEM((CHUNK, D), jnp.float32),
            pltpu.VMEM((CHUNK, D), jnp.float32),
            pltpu.VMEM((CHUNK, D), jnp.float32),
            pltpu.SemaphoreType.DMA,
            pltpu.SemaphoreType.DMA,
            pltpu.SemaphoreType.DMA,
            pltpu.SemaphoreType.DMA,
            pltpu.SemaphoreType.DMA,
            pltpu.SemaphoreType.DMA,
        ],
    )
    out = run(x2, table, pos_encoding)
    return out.reshape(B, S, D)
